# baseline (speedup 1.0000x reference)
import functools

import jax
import jax.numpy as jnp
from jax import lax
from jax.experimental import pallas as pl
from jax.experimental.pallas import tpu as pltpu
from jax.experimental.pallas import tpu_sc as plsc

N_EXP = 8
K = 2
HID = 1024
INTER = 2048
TOK = 2048
PAIRS = TOK * K

BLK = 128
PAD_ROWS = PAIRS + N_EXP * BLK
NB = PAD_ROWS // BLK


def _ffn_body(be_ref, x_ref, w0_ref, w1_ref, wo_ref, ws_ref, y_ref):
    x = x_ref[...]
    h0 = jnp.dot(x, w0_ref[0], preferred_element_type=jnp.float32)
    h1 = jnp.dot(x, w1_ref[0], preferred_element_type=jnp.float32)
    act = (h0 * jax.nn.sigmoid(h0)) * h1
    y = jnp.dot(act, wo_ref[0], preferred_element_type=jnp.float32)
    y_ref[...] = y * ws_ref[...]


def _grouped_ffn(x_sorted, wi_0, wi_1, wo, w_sorted, block_expert):
    grid_spec = pltpu.PrefetchScalarGridSpec(
        num_scalar_prefetch=1,
        grid=(NB,),
        in_specs=[
            pl.BlockSpec((BLK, HID), lambda b, be: (b, 0)),
            pl.BlockSpec((1, HID, INTER), lambda b, be: (be[b], 0, 0)),
            pl.BlockSpec((1, HID, INTER), lambda b, be: (be[b], 0, 0)),
            pl.BlockSpec((1, INTER, HID), lambda b, be: (be[b], 0, 0)),
            pl.BlockSpec((BLK, 1), lambda b, be: (b, 0)),
        ],
        out_specs=pl.BlockSpec((BLK, HID), lambda b, be: (b, 0)),
    )
    return pl.pallas_call(
        _ffn_body,
        grid_spec=grid_spec,
        out_shape=jax.ShapeDtypeStruct((PAD_ROWS, HID), jnp.float32),
    )(block_expert, x_sorted, wi_0, wi_1, wo, w_sorted.reshape(PAD_ROWS, 1))


NW = 32
_MESH = plsc.VectorSubcoreMesh(core_axis_name="c", subcore_axis_name="s")

ROWS_PER_W = PAD_ROWS // NW
GCHUNK = 40


def _sc_gather_x(x_hbm, idx_hbm, out_hbm, idx_v, rows_v, sem):
    wid = lax.axis_index("s") * 2 + lax.axis_index("c")
    base = wid * ROWS_PER_W
    nch = ROWS_PER_W // GCHUNK
    for c in range(nch):
        pltpu.sync_copy(idx_hbm.at[pl.ds(base + c * GCHUNK, GCHUNK)],
                        idx_v.at[c])
    for c in range(nch):
        for i in range(GCHUNK // 16):
            v = idx_v[c, pl.ds(i * 16, 16)]
            idx_v[c, pl.ds(i * 16, 16)] = jnp.clip(v, 0, TOK - 1)
    cps = [pltpu.make_async_copy(x_hbm.at[idx_v.at[c]], rows_v.at[c % 2], sem)
           for c in range(nch)]
    cps[0].start()
    for c in range(nch):
        cps[c].wait()
        if c + 1 < nch:
            cps[c + 1].start()
        pltpu.sync_copy(rows_v.at[c % 2],
                        out_hbm.at[pl.ds(base + c * GCHUNK, GCHUNK)])


def _gather_x(inputs, row_src):
    return pl.kernel(
        _sc_gather_x,
        out_type=jax.ShapeDtypeStruct((PAD_ROWS, HID), jnp.float32),
        mesh=_MESH,
        scratch_types=[
            pltpu.VMEM((ROWS_PER_W // GCHUNK, GCHUNK), jnp.int32),
            pltpu.VMEM((2, GCHUNK, HID), jnp.float32),
            pltpu.SemaphoreType.DMA,
        ],
    )(inputs, row_src)


TOK_PER_W = TOK // NW
CCHUNK = 32


def _sc_combine(y_hbm, pos0_hbm, pos1_hbm, out_hbm, idx0_v, idx1_v,
                buf0, buf1, obuf, sem):
    wid = lax.axis_index("s") * 2 + lax.axis_index("c")
    base = wid * TOK_PER_W
    nch = TOK_PER_W // CCHUNK
    for c in range(nch):
        pltpu.sync_copy(pos0_hbm.at[pl.ds(base + c * CCHUNK, CCHUNK)],
                        idx0_v.at[c])
        pltpu.sync_copy(pos1_hbm.at[pl.ds(base + c * CCHUNK, CCHUNK)],
                        idx1_v.at[c])
    for c in range(nch):
        cp0 = pltpu.make_async_copy(y_hbm.at[idx0_v.at[c]], buf0, sem)
        cp1 = pltpu.make_async_copy(y_hbm.at[idx1_v.at[c]], buf1, sem)
        cp0.start()
        cp1.start()
        cp0.wait()
        cp1.wait()
        for r in range(CCHUNK):
            def add_row(j, _, r=r):
                s = pl.ds(j * 16, 16)
                obuf[r, s] = buf0[r, s] + buf1[r, s]
                return _
            lax.fori_loop(0, HID // 16, add_row, 0)
        pltpu.sync_copy(obuf, out_hbm.at[pl.ds(base + c * CCHUNK, CCHUNK)])


def _combine(y, pos0, pos1):
    return pl.kernel(
        _sc_combine,
        out_type=jax.ShapeDtypeStruct((TOK, HID), jnp.float32),
        mesh=_MESH,
        scratch_types=[
            pltpu.VMEM((TOK_PER_W // CCHUNK, CCHUNK), jnp.int32),
            pltpu.VMEM((TOK_PER_W // CCHUNK, CCHUNK), jnp.int32),
            pltpu.VMEM((CCHUNK, HID), jnp.float32),
            pltpu.VMEM((CCHUNK, HID), jnp.float32),
            pltpu.VMEM((CCHUNK, HID), jnp.float32),
            pltpu.SemaphoreType.DMA,
        ],
    )(y, pos0, pos1)


def kernel(inputs, router_logits, wi_0, wi_1, wo):
    top_logits, top_idx = jax.lax.top_k(router_logits, K)
    w = jax.nn.softmax(top_logits.astype(jnp.float32), axis=-1)
    e_flat = top_idx.reshape(-1).astype(jnp.int32)
    w_flat = w.reshape(-1)

    sort_idx = jnp.argsort(e_flat, stable=True)
    e_sorted = e_flat[sort_idx]
    cnt = jnp.sum(jax.nn.one_hot(e_flat, N_EXP, dtype=jnp.int32), axis=0)
    padded = ((cnt + BLK - 1) // BLK) * BLK
    poff = jnp.concatenate([jnp.zeros(1, jnp.int32), jnp.cumsum(padded)])
    off = jnp.concatenate([jnp.zeros(1, jnp.int32), jnp.cumsum(cnt)])
    ranks = jnp.arange(PAIRS, dtype=jnp.int32)
    pos = poff[e_sorted] + (ranks - off[e_sorted])

    row_src = jnp.zeros(PAD_ROWS, jnp.int32).at[pos].set(sort_idx // K)
    w_sorted = jnp.zeros(PAD_ROWS, jnp.float32).at[pos].set(w_flat[sort_idx])
    pos_pair = jnp.zeros(PAIRS, jnp.int32).at[sort_idx].set(pos)
    pos0 = pos_pair[0::K]
    pos1 = pos_pair[1::K]

    block_rows = jnp.arange(NB, dtype=jnp.int32) * BLK
    block_expert = jnp.clip(
        jnp.searchsorted(poff, block_rows, side="right").astype(jnp.int32) - 1,
        0, N_EXP - 1)

    x_sorted = _gather_x(inputs, row_src)
    y = _grouped_ffn(x_sorted, wi_0, wi_1, wo, w_sorted, block_expert)
    return _combine(y, pos0, pos1)
